# baseline (speedup 1.0000x reference)
import functools

import jax
import jax.numpy as jnp
from jax import lax
from jax.experimental import pallas as pl
from jax.experimental.pallas import tpu as pltpu
from jax.experimental.pallas import tpu_sc as plsc

BATCH = 4096
HIST = 200
EMBED = 64
B = BATCH * HIST
VOCAB = 1000000
VPAD = 1000064

NC = 2
NS = 16
NW = NC * NS

_mesh = plsc.VectorSubcoreMesh(core_axis_name="c", subcore_axis_name="s")

NVB = VPAD // 128
NI1 = 246


@functools.partial(
    pl.kernel,
    mesh=_mesh,
    out_type=jax.ShapeDtypeStruct((VPAD // 2, 128), jnp.float32),
    scratch_types=[
        pltpu.VMEM((2, EMBED, 128), jnp.float32),
        pltpu.VMEM((2, EMBED, 128), jnp.float32),
        pltpu.SemaphoreType.DMA,
        pltpu.SemaphoreType.DMA,
        pltpu.SemaphoreType.DMA,
        pltpu.SemaphoreType.DMA,
    ],
    compiler_params=pltpu.CompilerParams(use_tc_tiling_on_sc=True, needs_layout_passes=False),
)
def _relayout(tabT, rows_hbm, strip_v, pair_v, g0, g1, o0, o1):
    wid = lax.axis_index("s") * NC + lax.axis_index("c")

    def vb_of(i):
        vb = wid + i * NW
        return jnp.where(vb < NVB, vb, wid)

    def fire_in(i, half, sem):
        pltpu.async_copy(
            tabT.at[:, pl.ds(vb_of(i) * 128, 128)], strip_v.at[half], sem
        )

    def wait_in(half, sem):
        pltpu.make_async_copy(
            tabT.at[:, pl.ds(0, 128)], strip_v.at[half], sem
        ).wait()

    def transpose(half):
        lanes = lax.iota(jnp.int32, 16)

        rbase = (lanes & 1) * EMBED

        @plsc.parallel_loop(0, 32, unroll=2)
        def _(s):
            g = s >> 2
            c = s & 3
            vlvec = g * 16 + lanes
            qvec = g * 8 + (lanes >> 1)
            c16 = c * 16
            for j in range(16):
                dvec = ((lanes + j) & 15) + c16
                vals = plsc.load_gather(strip_v.at[half], [dvec, vlvec])
                plsc.store_scatter(
                    pair_v.at[half], [qvec, rbase + dvec], vals
                )

    def fire_out(i, half, sem):
        pltpu.async_copy(
            pair_v.at[half], rows_hbm.at[pl.ds(vb_of(i) * EMBED, EMBED)], sem
        )

    def wait_out(half, sem):
        pltpu.make_async_copy(
            pair_v.at[half], rows_hbm.at[pl.ds(0, EMBED)], sem
        ).wait()

    fire_in(0, 0, g0)

    def body(t, carry):
        i0 = 2 * t
        i1 = i0 + 1
        fire_in(i1, 1, g1)
        wait_in(0, g0)

        @pl.when(t > 0)
        def _():
            wait_out(0, o0)

        transpose(0)
        fire_out(i0, 0, o0)

        @pl.when(i1 + 1 < NI1)
        def _():
            fire_in(i1 + 1, 0, g0)

        wait_in(1, g1)

        @pl.when(t > 0)
        def _():
            wait_out(1, o1)

        transpose(1)
        fire_out(i1, 1, o1)
        return carry

    lax.fori_loop(0, NI1 // 2, body, 0)
    wait_out(0, o0)
    wait_out(1, o1)


CHUNK = 128
B_PER_W = B // NW
NCH = B_PER_W // CHUNK
BHI = BATCH // 128


@functools.partial(
    pl.kernel,
    mesh=_mesh,
    out_type=jax.ShapeDtypeStruct((HIST, 8, BHI, 8, 128), jnp.float32),
    scratch_types=[
        pltpu.VMEM((B_PER_W,), jnp.int32),
        pltpu.VMEM((2, CHUNK, EMBED), jnp.float32),
        pltpu.VMEM((2, 8, 8, 133), jnp.float32),
        pltpu.SemaphoreType.DMA,
        pltpu.SemaphoreType.DMA,
        pltpu.SemaphoreType.DMA,
        pltpu.SemaphoreType.DMA,
    ],
    compiler_params=pltpu.CompilerParams(use_tc_tiling_on_sc=False, needs_layout_passes=False),
)
def _gather(idx_hbm, rows_hbm, out5, idx_v, rows_v, ob_v, g0, g1, o0, o1):
    wid = lax.axis_index("s") * NC + lax.axis_index("c")
    k0 = wid * NCH

    pltpu.sync_copy(idx_hbm.at[pl.ds(k0 * CHUNK, B_PER_W)], idx_v)

    def fire_g(j, half, sem):
        pltpu.async_copy(
            rows_hbm.at[idx_v.at[pl.ds(j * CHUNK, CHUNK)]],
            rows_v.at[half],
            sem,
        )

    def wait_g(half, sem):
        pltpu.make_async_copy(
            rows_hbm.at[pl.ds(0, CHUNK)], rows_v.at[half], sem
        ).wait()

    lanes = lax.iota(jnp.int32, 16)
    dhi_c = [(c * 16 + lanes) >> 3 for c in range(4)]
    dlo_c = [(c * 16 + lanes) & 7 for c in range(4)]

    def transpose(half):
        @plsc.parallel_loop(0, CHUNK, unroll=4)
        def _(b):
            bvec = jnp.full((16,), 0, jnp.int32) + b
            for c in range(4):
                vals = rows_v[half, b, pl.ds(c * 16, 16)]
                plsc.store_scatter(
                    ob_v.at[half], [dhi_c[c], dlo_c[c], bvec], vals
                )

    def fire_o(j, half, sem):
        k = k0 + j
        pltpu.async_copy(
            ob_v.at[half, :, :, pl.ds(0, 128)],
            out5.at[k // BHI, slice(None), k % BHI],
            sem,
        )

    def wait_o(half, sem):
        pltpu.make_async_copy(
            ob_v.at[half, :, :, pl.ds(0, 128)],
            out5.at[0, slice(None), 0],
            sem,
        ).wait()

    fire_g(0, 0, g0)

    def body(t, carry):
        j0 = 2 * t
        j1 = j0 + 1
        fire_g(j1, 1, g1)
        wait_g(0, g0)

        @pl.when(t > 0)
        def _():
            wait_o(0, o0)

        transpose(0)
        fire_o(j0, 0, o0)

        @pl.when(j1 + 1 < NCH)
        def _():
            fire_g(j1 + 1, 0, g0)

        wait_g(1, g1)

        @pl.when(t > 0)
        def _():
            wait_o(1, o1)

        transpose(1)
        fire_o(j1, 1, o1)
        return carry

    lax.fori_loop(0, NCH // 2, body, 0)
    wait_o(0, o0)
    wait_o(1, o1)


def kernel(x, table):
    rows2 = _relayout(table.T)
    rows = rows2.reshape(VPAD, EMBED)
    idx = x.T.reshape(B).astype(jnp.int32)
    out5 = _gather(idx, rows)
    return out5.transpose(2, 4, 0, 1, 3).reshape(BATCH, HIST, EMBED)


# speedup vs baseline: 1.1075x; 1.1075x over previous
import functools

import jax
import jax.numpy as jnp
from jax import lax
from jax.experimental import pallas as pl
from jax.experimental.pallas import tpu as pltpu
from jax.experimental.pallas import tpu_sc as plsc

BATCH = 4096
HIST = 200
EMBED = 64
B = BATCH * HIST
VOCAB = 1000000
VPAD = 1000064

NC = 2
NS = 16
NW = NC * NS

_mesh = plsc.VectorSubcoreMesh(core_axis_name="c", subcore_axis_name="s")

NVB = VPAD // 128
NI1 = 246


@functools.partial(
    pl.kernel,
    mesh=_mesh,
    out_type=jax.ShapeDtypeStruct((VPAD // 2, 128), jnp.float32),
    scratch_types=[
        pltpu.VMEM((2, EMBED, 128), jnp.float32),
        pltpu.VMEM((2, EMBED, 128), jnp.float32),
        pltpu.SemaphoreType.DMA,
        pltpu.SemaphoreType.DMA,
        pltpu.SemaphoreType.DMA,
        pltpu.SemaphoreType.DMA,
    ],
    compiler_params=pltpu.CompilerParams(use_tc_tiling_on_sc=True, needs_layout_passes=False),
)
def _relayout(tabT, rows_hbm, strip_v, pair_v, g0, g1, o0, o1):
    wid = lax.axis_index("s") * NC + lax.axis_index("c")

    def vb_of(i):
        vb = wid + i * NW
        return jnp.where(vb < NVB, vb, wid)

    def fire_in(i, half, sem):
        pltpu.async_copy(
            tabT.at[:, pl.ds(vb_of(i) * 128, 128)], strip_v.at[half], sem
        )

    def wait_in(half, sem):
        pltpu.make_async_copy(
            tabT.at[:, pl.ds(0, 128)], strip_v.at[half], sem
        ).wait()

    def transpose(half):
        lanes = lax.iota(jnp.int32, 16)

        @plsc.parallel_loop(0, 32, unroll=2)
        def _(s):
            g = s >> 2
            c = s & 3
            vlvec = g * 16 + lanes
            vlbase = vlvec * EMBED
            c16 = c * 16
            for j in range(16):
                dvec = ((lanes + j) & 15) + c16
                flat = vlbase + dvec
                vals = plsc.load_gather(strip_v.at[half], [dvec, vlvec])
                plsc.store_scatter(
                    pair_v.at[half], [flat >> 7, flat & 127], vals
                )

    def fire_out(i, half, sem):
        pltpu.async_copy(
            pair_v.at[half], rows_hbm.at[pl.ds(vb_of(i) * EMBED, EMBED)], sem
        )

    def wait_out(half, sem):
        pltpu.make_async_copy(
            pair_v.at[half], rows_hbm.at[pl.ds(0, EMBED)], sem
        ).wait()

    fire_in(0, 0, g0)

    def body(t, carry):
        i0 = 2 * t
        i1 = i0 + 1
        fire_in(i1, 1, g1)
        wait_in(0, g0)

        @pl.when(t > 0)
        def _():
            wait_out(0, o0)

        transpose(0)
        fire_out(i0, 0, o0)

        @pl.when(i1 + 1 < NI1)
        def _():
            fire_in(i1 + 1, 0, g0)

        wait_in(1, g1)

        @pl.when(t > 0)
        def _():
            wait_out(1, o1)

        transpose(1)
        fire_out(i1, 1, o1)
        return carry

    lax.fori_loop(0, NI1 // 2, body, 0)
    wait_out(0, o0)
    wait_out(1, o1)


CHUNK = 128
B_PER_W = B // NW
NCH = B_PER_W // CHUNK
BHI = BATCH // 128


@functools.partial(
    pl.kernel,
    mesh=_mesh,
    out_type=jax.ShapeDtypeStruct((HIST, 8, BHI, 8, 128), jnp.float32),
    scratch_types=[
        pltpu.VMEM((B_PER_W,), jnp.int32),
        pltpu.VMEM((4, CHUNK, EMBED), jnp.float32),
        pltpu.VMEM((4, 8, 8, 133), jnp.float32),
        pltpu.SemaphoreType.DMA,
        pltpu.SemaphoreType.DMA,
        pltpu.SemaphoreType.DMA,
        pltpu.SemaphoreType.DMA,
        pltpu.SemaphoreType.DMA,
        pltpu.SemaphoreType.DMA,
        pltpu.SemaphoreType.DMA,
        pltpu.SemaphoreType.DMA,
    ],
    compiler_params=pltpu.CompilerParams(use_tc_tiling_on_sc=False, needs_layout_passes=False),
)
def _gather(idx_hbm, rows_hbm, out5, idx_v, rows_v, ob_v,
            g0, g1, g2, g3, o0, o1, o2, o3):
    gs = (g0, g1, g2, g3)
    os = (o0, o1, o2, o3)
    wid = lax.axis_index("s") * NC + lax.axis_index("c")
    k0 = wid * NCH

    pltpu.sync_copy(idx_hbm.at[pl.ds(k0 * CHUNK, B_PER_W)], idx_v)

    def fire_g(j, half, sem):
        pltpu.async_copy(
            rows_hbm.at[idx_v.at[pl.ds(j * CHUNK, CHUNK)]],
            rows_v.at[half],
            sem,
        )

    def wait_g(half, sem):
        pltpu.make_async_copy(
            rows_hbm.at[pl.ds(0, CHUNK)], rows_v.at[half], sem
        ).wait()

    lanes = lax.iota(jnp.int32, 16)
    dhi_c = [(c * 16 + lanes) >> 3 for c in range(4)]
    dlo_c = [(c * 16 + lanes) & 7 for c in range(4)]

    def transpose(half):
        @plsc.parallel_loop(0, CHUNK, unroll=4)
        def _(b):
            bvec = jnp.full((16,), 0, jnp.int32) + b
            for c in range(4):
                vals = rows_v[half, b, pl.ds(c * 16, 16)]
                plsc.store_scatter(
                    ob_v.at[half], [dhi_c[c], dlo_c[c], bvec], vals
                )

    def fire_o(j, half, sem):
        k = k0 + j
        pltpu.async_copy(
            ob_v.at[half, :, :, pl.ds(0, 128)],
            out5.at[k // BHI, slice(None), k % BHI],
            sem,
        )

    def wait_o(half, sem):
        pltpu.make_async_copy(
            ob_v.at[half, :, :, pl.ds(0, 128)],
            out5.at[0, slice(None), 0],
            sem,
        ).wait()

    for b in range(4):
        fire_g(b, b, gs[b])

    def body(t, carry):
        for b in range(4):
            j = 4 * t + b
            wait_g(b, gs[b])

            @pl.when(t > 0)
            def _():
                wait_o(b, os[b])

            transpose(b)
            fire_o(j, b, os[b])

            @pl.when(j + 4 < NCH)
            def _():
                fire_g(j + 4, b, gs[b])

        return carry

    lax.fori_loop(0, NCH // 4, body, 0)
    for b in range(4):
        wait_o(b, os[b])


def kernel(x, table):
    rows2 = _relayout(table.T)
    rows = rows2.reshape(VPAD, EMBED)
    idx = x.T.reshape(B).astype(jnp.int32)
    out5 = _gather(idx, rows)
    return out5.transpose(2, 4, 0, 1, 3).reshape(BATCH, HIST, EMBED)


# speedup vs baseline: 1.2979x; 1.1719x over previous
import functools

import jax
import jax.numpy as jnp
from jax import lax
from jax.experimental import pallas as pl
from jax.experimental.pallas import tpu as pltpu
from jax.experimental.pallas import tpu_sc as plsc

BATCH = 4096
HIST = 200
EMBED = 64
B = BATCH * HIST
VOCAB = 1000000
VPAD = 1000064

NC = 2
NS = 16
NW = NC * NS

_mesh = plsc.VectorSubcoreMesh(core_axis_name="c", subcore_axis_name="s")

NVB = VPAD // 128
NI1 = 248


@functools.partial(
    pl.kernel,
    mesh=_mesh,
    out_type=jax.ShapeDtypeStruct((VPAD // 2, 128), jnp.float32),
    scratch_types=[
        pltpu.VMEM((4, EMBED, 128), jnp.float32),
        pltpu.VMEM((4, EMBED, 128), jnp.float32),
        pltpu.SemaphoreType.DMA,
        pltpu.SemaphoreType.DMA,
        pltpu.SemaphoreType.DMA,
        pltpu.SemaphoreType.DMA,
        pltpu.SemaphoreType.DMA,
        pltpu.SemaphoreType.DMA,
        pltpu.SemaphoreType.DMA,
        pltpu.SemaphoreType.DMA,
    ],
    compiler_params=pltpu.CompilerParams(use_tc_tiling_on_sc=True, needs_layout_passes=False),
)
def _relayout(tabT, rows_hbm, strip_v, pair_v,
              g0, g1, g2, g3, o0, o1, o2, o3):
    gs = (g0, g1, g2, g3)
    os = (o0, o1, o2, o3)
    wid = lax.axis_index("s") * NC + lax.axis_index("c")

    def vb_of(i):
        vb = wid + i * NW
        return jnp.where(vb < NVB, vb, wid)

    def fire_in(i, half, sem):
        pltpu.async_copy(
            tabT.at[:, pl.ds(vb_of(i) * 128, 128)], strip_v.at[half], sem
        )

    def wait_in(half, sem):
        pltpu.make_async_copy(
            tabT.at[:, pl.ds(0, 128)], strip_v.at[half], sem
        ).wait()

    def transpose(half):
        lanes = lax.iota(jnp.int32, 16)

        @plsc.parallel_loop(0, 32, unroll=2)
        def _(s):
            g = s >> 2
            c = s & 3
            vlvec = g * 16 + lanes
            vlbase = vlvec * EMBED
            c16 = c * 16
            for j in range(16):
                dvec = ((lanes + j) & 15) + c16
                flat = vlbase + dvec
                vals = plsc.load_gather(strip_v.at[half], [dvec, vlvec])
                plsc.store_scatter(
                    pair_v.at[half], [flat >> 7, flat & 127], vals
                )

    def fire_out(i, half, sem):
        pltpu.async_copy(
            pair_v.at[half], rows_hbm.at[pl.ds(vb_of(i) * EMBED, EMBED)], sem
        )

    def wait_out(half, sem):
        pltpu.make_async_copy(
            pair_v.at[half], rows_hbm.at[pl.ds(0, EMBED)], sem
        ).wait()

    for b in range(4):
        fire_in(b, b, gs[b])

    def body(t, carry):
        for b in range(4):
            i = 4 * t + b
            wait_in(b, gs[b])

            @pl.when(t > 0)
            def _():
                wait_out(b, os[b])

            transpose(b)
            fire_out(i, b, os[b])

            @pl.when(i + 4 < NI1)
            def _():
                fire_in(i + 4, b, gs[b])

        return carry

    lax.fori_loop(0, NI1 // 4, body, 0)
    for b in range(4):
        wait_out(b, os[b])


CHUNK = 128
B_PER_W = B // NW
NCH = B_PER_W // CHUNK
BHI = BATCH // 128


@functools.partial(
    pl.kernel,
    mesh=_mesh,
    out_type=jax.ShapeDtypeStruct((HIST, 8, BHI, 8, 128), jnp.float32),
    scratch_types=[
        pltpu.VMEM((B_PER_W,), jnp.int32),
        pltpu.VMEM((4, CHUNK, EMBED), jnp.float32),
        pltpu.VMEM((4, 8, 8, 133), jnp.float32),
        pltpu.SemaphoreType.DMA,
        pltpu.SemaphoreType.DMA,
        pltpu.SemaphoreType.DMA,
        pltpu.SemaphoreType.DMA,
        pltpu.SemaphoreType.DMA,
        pltpu.SemaphoreType.DMA,
        pltpu.SemaphoreType.DMA,
        pltpu.SemaphoreType.DMA,
    ],
    compiler_params=pltpu.CompilerParams(use_tc_tiling_on_sc=False, needs_layout_passes=False),
)
def _gather(idx_hbm, rows_hbm, out5, idx_v, rows_v, ob_v,
            g0, g1, g2, g3, o0, o1, o2, o3):
    gs = (g0, g1, g2, g3)
    os = (o0, o1, o2, o3)
    wid = lax.axis_index("s") * NC + lax.axis_index("c")
    k0 = wid * NCH

    pltpu.sync_copy(idx_hbm.at[pl.ds(k0 * CHUNK, B_PER_W)], idx_v)

    def fire_g(j, half, sem):
        pltpu.async_copy(
            rows_hbm.at[idx_v.at[pl.ds(j * CHUNK, CHUNK)]],
            rows_v.at[half],
            sem,
        )

    def wait_g(half, sem):
        pltpu.make_async_copy(
            rows_hbm.at[pl.ds(0, CHUNK)], rows_v.at[half], sem
        ).wait()

    lanes = lax.iota(jnp.int32, 16)
    dhi_c = [(c * 16 + lanes) >> 3 for c in range(4)]
    dlo_c = [(c * 16 + lanes) & 7 for c in range(4)]

    def transpose(half):
        @plsc.parallel_loop(0, CHUNK, unroll=4)
        def _(b):
            bvec = jnp.full((16,), 0, jnp.int32) + b
            for c in range(4):
                vals = rows_v[half, b, pl.ds(c * 16, 16)]
                plsc.store_scatter(
                    ob_v.at[half], [dhi_c[c], dlo_c[c], bvec], vals
                )

    def fire_o(j, half, sem):
        k = k0 + j
        pltpu.async_copy(
            ob_v.at[half, :, :, pl.ds(0, 128)],
            out5.at[k // BHI, slice(None), k % BHI],
            sem,
        )

    def wait_o(half, sem):
        pltpu.make_async_copy(
            ob_v.at[half, :, :, pl.ds(0, 128)],
            out5.at[0, slice(None), 0],
            sem,
        ).wait()

    for b in range(4):
        fire_g(b, b, gs[b])

    def body(t, carry):
        for b in range(4):
            j = 4 * t + b
            wait_g(b, gs[b])

            @pl.when(t > 0)
            def _():
                wait_o(b, os[b])

            transpose(b)
            fire_o(j, b, os[b])

            @pl.when(j + 4 < NCH)
            def _():
                fire_g(j + 4, b, gs[b])

        return carry

    lax.fori_loop(0, NCH // 4, body, 0)
    for b in range(4):
        wait_o(b, os[b])


def kernel(x, table):
    rows2 = _relayout(table.T)
    rows = rows2.reshape(VPAD, EMBED)
    idx = x.T.reshape(B).astype(jnp.int32)
    out5 = _gather(idx, rows)
    return out5.transpose(2, 4, 0, 1, 3).reshape(BATCH, HIST, EMBED)
